# baseline (speedup 1.0000x reference)
import functools

import jax
import jax.numpy as jnp
from jax import lax
from jax.experimental import pallas as pl
from jax.experimental.pallas import tpu as pltpu
from jax.experimental.pallas import tpu_sc as plsc

B = 4096
L = 200
EMB = 64
DMODEL = 128
VOCAB = 1000000
N = B * L

NC = 2
NS = 16
NW = NC * NS

SEQ_PER_W = B // NW
IDX_SPLITS = ((0, 128), (128, 72))

VB = 8192


def _tc_project_table(tableT, W, b2):
    def body(t_ref, w_ref, b_ref, o_ref):
        tp = lax.dot_general(
            t_ref[...], w_ref[...],
            (((0,), (0,)), ((), ())),
            preferred_element_type=jnp.float32,
        )
        o_ref[...] = tp + b_ref[...]

    return pl.pallas_call(
        body,
        grid=(pl.cdiv(VOCAB, VB),),
        in_specs=[
            pl.BlockSpec((EMB, VB), lambda i: (0, i)),
            pl.BlockSpec((EMB, DMODEL), lambda i: (0, 0)),
            pl.BlockSpec((1, DMODEL), lambda i: (0, 0)),
        ],
        out_specs=pl.BlockSpec((VB, DMODEL), lambda i: (i, 0)),
        out_shape=jax.ShapeDtypeStruct((VOCAB, DMODEL), jnp.float32),
        compiler_params=pltpu.CompilerParams(
            dimension_semantics=("arbitrary",),
        ),
    )(tableT, W, b2)


NBUF = 4


def _make_sc_gather():
    mesh = plsc.VectorSubcoreMesh(core_axis_name="c", subcore_axis_name="s")
    scratch = (
        [pltpu.VMEM((L,), jnp.int32) for _ in range(NBUF)]
        + [pltpu.VMEM((L, DMODEL), jnp.float32) for _ in range(NBUF)]
        + [pltpu.SemaphoreType.DMA((NBUF,)) for _ in range(4)]
    )

    @functools.partial(
        pl.kernel,
        out_type=jax.ShapeDtypeStruct((N, DMODEL), jnp.float32),
        mesh=mesh,
        scratch_types=scratch,
    )
    def sc_gather(idx_hbm, tproj_hbm, pe_hbm, out_hbm, *scr):
        idx_v = scr[0:NBUF]
        rows_v = scr[NBUF : 2 * NBUF]
        sem_idx, sem_pe, sem_g, sem_wb = scr[2 * NBUF :]
        wid = lax.axis_index("s") * NC + lax.axis_index("c")
        seq0 = wid * SEQ_PER_W

        def start_init(b, s):
            t0 = (seq0 + s) * L
            pltpu.async_copy(idx_hbm.at[pl.ds(t0, L)], idx_v[b], sem_idx.at[b])
            pltpu.async_copy(pe_hbm, rows_v[b], sem_pe.at[b])

        for b in range(NBUF - 1):
            start_init(b, b)

        def group(g0, carry):
            for b in range(NBUF):
                s = g0 * NBUF + b
                t0 = (seq0 + s) * L
                pltpu.make_async_copy(
                    idx_hbm.at[pl.ds(t0, L)], idx_v[b], sem_idx.at[b]
                ).wait()
                pltpu.make_async_copy(pe_hbm, rows_v[b], sem_pe.at[b]).wait()
                for off, sz in IDX_SPLITS:
                    pltpu.async_copy(
                        tproj_hbm.at[idx_v[b].at[pl.ds(off, sz)]],
                        rows_v[b].at[pl.ds(off, sz)],
                        sem_g.at[b],
                        add=True,
                    )
                bp = (b - 1) % NBUF
                s_re = s + NBUF - 1

                @pl.when(s_re < SEQ_PER_W)
                def _():
                    t0p = (seq0 + s_re) * L

                    @pl.when(s_re >= NBUF)
                    def _():
                        pltpu.make_async_copy(
                            rows_v[bp], out_hbm.at[pl.ds(t0p, L)], sem_wb.at[bp]
                        ).wait()

                    start_init(bp, s_re)

                for off, sz in IDX_SPLITS:
                    pltpu.make_async_copy(
                        tproj_hbm.at[idx_v[b].at[pl.ds(off, sz)]],
                        rows_v[b].at[pl.ds(off, sz)],
                        sem_g.at[b],
                    ).wait()
                pltpu.async_copy(rows_v[b], out_hbm.at[pl.ds(t0, L)], sem_wb.at[b])
            return carry

        lax.fori_loop(0, SEQ_PER_W // NBUF, group, 0)
        for b in range(NBUF):
            t0 = (seq0 + SEQ_PER_W - NBUF + b) * L
            pltpu.make_async_copy(
                rows_v[b], out_hbm.at[pl.ds(t0, L)], sem_wb.at[b]
            ).wait()

    return sc_gather


_sc_gather = _make_sc_gather()


@jax.jit
def kernel(x, table, W, b, pe):
    tproj = _tc_project_table(table.T, W, b.reshape(1, DMODEL))
    out = _sc_gather(x.reshape(N), tproj, pe[0, :L])
    return out.reshape(B, L, DMODEL)


# speedup vs baseline: 1.7279x; 1.7279x over previous
import functools

import jax
import jax.numpy as jnp
from jax import lax
from jax.experimental import pallas as pl
from jax.experimental.pallas import tpu as pltpu
from jax.experimental.pallas import tpu_sc as plsc

B = 4096
L = 200
EMB = 64
DMODEL = 128
VOCAB = 1000000
N = B * L

NC = 2
NS = 16
NW = NC * NS

SEQ_PER_W = B // NW
IDX_SPLITS = ((0, 128), (128, 72))

VB = 8192


def _tc_project_table(tableT, W, b2):
    def body(t_ref, w_ref, b_ref, o_ref):
        tp = lax.dot_general(
            t_ref[...], w_ref[...],
            (((0,), (0,)), ((), ())),
            preferred_element_type=jnp.float32,
        )
        o_ref[...] = tp + b_ref[...]

    return pl.pallas_call(
        body,
        grid=(pl.cdiv(VOCAB, VB),),
        in_specs=[
            pl.BlockSpec((EMB, VB), lambda i: (0, i)),
            pl.BlockSpec((EMB, DMODEL), lambda i: (0, 0)),
            pl.BlockSpec((1, DMODEL), lambda i: (0, 0)),
        ],
        out_specs=pl.BlockSpec((VB, DMODEL), lambda i: (i, 0)),
        out_shape=jax.ShapeDtypeStruct((VOCAB, DMODEL), jnp.float32),
        compiler_params=pltpu.CompilerParams(
            dimension_semantics=("arbitrary",),
        ),
    )(tableT, W, b2)


NBUF = 4


def _make_sc_gather():
    mesh = plsc.VectorSubcoreMesh(core_axis_name="c", subcore_axis_name="s")
    scratch = (
        [pltpu.VMEM((L,), jnp.int32) for _ in range(NBUF)]
        + [pltpu.VMEM((L, DMODEL), jnp.float32) for _ in range(NBUF)]
        + [pltpu.SemaphoreType.DMA((NBUF,)) for _ in range(4)]
    )

    @functools.partial(
        pl.kernel,
        out_type=jax.ShapeDtypeStruct((N, DMODEL), jnp.float32),
        mesh=mesh,
        scratch_types=scratch,
    )
    def sc_gather(idx_hbm, tproj_hbm, pe_hbm, out_hbm, *scr):
        idx_v = scr[0:NBUF]
        rows_v = scr[NBUF : 2 * NBUF]
        sem_idx, sem_pe, sem_g, sem_wb = scr[2 * NBUF :]
        wid = lax.axis_index("s") * NC + lax.axis_index("c")
        seq0 = wid * SEQ_PER_W

        pe_base = wid * L

        def start_init(b, s):
            t0 = (seq0 + s) * L
            pltpu.async_copy(idx_hbm.at[pl.ds(t0, L)], idx_v[b], sem_idx.at[b])
            pltpu.async_copy(
                pe_hbm.at[pl.ds(pe_base, L)], rows_v[b], sem_pe.at[b]
            )

        for b in range(NBUF - 1):
            start_init(b, b)

        def group(g0, carry):
            for b in range(NBUF):
                s = g0 * NBUF + b
                t0 = (seq0 + s) * L
                pltpu.make_async_copy(
                    idx_hbm.at[pl.ds(t0, L)], idx_v[b], sem_idx.at[b]
                ).wait()
                pltpu.make_async_copy(
                    pe_hbm.at[pl.ds(pe_base, L)], rows_v[b], sem_pe.at[b]
                ).wait()
                for off, sz in IDX_SPLITS:
                    pltpu.async_copy(
                        tproj_hbm.at[idx_v[b].at[pl.ds(off, sz)]],
                        rows_v[b].at[pl.ds(off, sz)],
                        sem_g.at[b],
                        add=True,
                    )
                bp = (b - 1) % NBUF
                s_re = s + NBUF - 1

                @pl.when(s_re < SEQ_PER_W)
                def _():
                    t0p = (seq0 + s_re) * L

                    @pl.when(s_re >= NBUF)
                    def _():
                        pltpu.make_async_copy(
                            rows_v[bp], out_hbm.at[pl.ds(t0p, L)], sem_wb.at[bp]
                        ).wait()

                    start_init(bp, s_re)

                for off, sz in IDX_SPLITS:
                    pltpu.make_async_copy(
                        tproj_hbm.at[idx_v[b].at[pl.ds(off, sz)]],
                        rows_v[b].at[pl.ds(off, sz)],
                        sem_g.at[b],
                    ).wait()
                pltpu.async_copy(rows_v[b], out_hbm.at[pl.ds(t0, L)], sem_wb.at[b])
            return carry

        lax.fori_loop(0, SEQ_PER_W // NBUF, group, 0)
        for b in range(NBUF):
            t0 = (seq0 + SEQ_PER_W - NBUF + b) * L
            pltpu.make_async_copy(
                rows_v[b], out_hbm.at[pl.ds(t0, L)], sem_wb.at[b]
            ).wait()

    return sc_gather


_sc_gather = _make_sc_gather()


@jax.jit
def kernel(x, table, W, b, pe):
    tproj = _tc_project_table(table.T, W, b.reshape(1, DMODEL))
    pe_rep = jnp.tile(pe[0, :L], (NW, 1))
    out = _sc_gather(x.reshape(N), tproj, pe_rep)
    return out.reshape(B, L, DMODEL)
